# Initial kernel scaffold; baseline (speedup 1.0000x reference)
import jax
import jax.numpy as jnp
from jax.experimental import pallas as pl


def kernel(user, item_seq, user_recipe_idx, recipe_user_idx, recipe_recipe_idx, recipe_ing_idx, user_embedding, recipe_embedding, ing_embedding, Wself_u, Wnei_u, a_u, Wself_ru, Wnei_ru, a_ru, Wself_rr, Wnei_rr, a_rr, Wself_ri, Wnei_ri, a_ri, W_u, b_u, W_r, b_r, W_rel, q_rel):
    raise NotImplementedError("write your pallas kernel here")


# speedup vs baseline: 1.4573x; 1.4573x over previous
import functools

import jax
import jax.numpy as jnp
from jax import lax
from jax.experimental import pallas as pl
from jax.experimental.pallas import tpu as pltpu
from jax.experimental.pallas import tpu_sc as plsc

N_USER = 10000
N_RECIPE = 5900
N_ING = 6000
D = 128
H = 4
OUT = 128
B = 1024
DEG = 16

NR_PAD = 6144
E_REL = NR_PAD * DEG
E_U = B * DEG

_NC, _NS = 2, 16
_NW = _NC * _NS
_CH = 128


def _widx():
    return lax.axis_index("s") * _NC + lax.axis_index("c")


def _sc_nested(user, user_recipe_idx_pad, user_embedding):
    per_w = B // _NW

    mesh = plsc.VectorSubcoreMesh(core_axis_name="c", subcore_axis_name="s")

    @functools.partial(
        pl.kernel,
        out_type=(
            jax.ShapeDtypeStruct((B, 128), jnp.int32),
            jax.ShapeDtypeStruct((B, D), jnp.float32),
        ),
        mesh=mesh,
        scratch_types=[
            pltpu.VMEM((per_w,), jnp.int32),
            pltpu.VMEM((per_w, 128), jnp.int32),
            pltpu.VMEM((per_w, D), jnp.float32),
            pltpu.SemaphoreType.DMA,
            pltpu.SemaphoreType.DMA,
        ],
    )
    def k(user_h, uri_h, uemb_h, idx_out_h, hu_out_h, uid_v, urow_v, emb_v, s1, s2):
        wid = _widx()
        base = wid * per_w
        pltpu.sync_copy(user_h.at[pl.ds(base, per_w)], uid_v)
        c1 = pltpu.async_copy(uri_h.at[uid_v], urow_v, s1)
        c2 = pltpu.async_copy(uemb_h.at[uid_v], emb_v, s2)
        c1.wait()
        pltpu.sync_copy(urow_v, idx_out_h.at[pl.ds(base, per_w)])
        c2.wait()
        pltpu.sync_copy(emb_v, hu_out_h.at[pl.ds(base, per_w)])

    return k(user, user_recipe_idx_pad, user_embedding)


def _sc_gather_rows(idx_ru, idx_rr, idx_ri, idx_un,
                    user_embedding, recipe_embedding, ing_embedding):
    mesh = plsc.VectorSubcoreMesh(core_axis_name="c", subcore_axis_name="s")

    @functools.partial(
        pl.kernel,
        out_type=(
            jax.ShapeDtypeStruct((E_REL, D), jnp.float32),
            jax.ShapeDtypeStruct((E_REL, D), jnp.float32),
            jax.ShapeDtypeStruct((E_REL, D), jnp.float32),
            jax.ShapeDtypeStruct((E_U, D), jnp.float32),
        ),
        mesh=mesh,
        scratch_types=[
            pltpu.VMEM((_CH,), jnp.int32),
            pltpu.VMEM((_CH, D), jnp.float32),
            pltpu.SemaphoreType.DMA,
        ],
    )
    def k(iru_h, irr_h, iri_h, iun_h, uemb_h, remb_h, iemb_h,
          oru_h, orr_h, ori_h, oun_h, idx_v, rows_v, sem):
        wid = _widx()

        def task(idx_h, table_h, out_h, n_edges):
            per_w = n_edges // _NW
            nch = per_w // _CH

            def body(c, carry):
                base = wid * per_w + c * _CH
                pltpu.sync_copy(idx_h.at[pl.ds(base, _CH)], idx_v)
                pltpu.async_copy(table_h.at[idx_v], rows_v, sem).wait()
                pltpu.sync_copy(rows_v, out_h.at[pl.ds(base, _CH)])
                return carry

            lax.fori_loop(0, nch, body, 0)

        task(iru_h, uemb_h, oru_h, E_REL)
        task(irr_h, remb_h, orr_h, E_REL)
        task(iri_h, iemb_h, ori_h, E_REL)
        task(iun_h, remb_h, oun_h, E_U)

    return k(idx_ru, idx_rr, idx_ri, idx_un,
             user_embedding, recipe_embedding, ing_embedding)


def _attend(h_self, nei3, Wself, Wnei, a):
    outs = []
    for h in range(H):
        a1 = a[h:h + 1, :OUT]
        a2 = a[h:h + 1, OUT:]
        Uh = jnp.sum(Wself[h] * a1, axis=1)
        Vh = jnp.sum(Wnei[h] * a2, axis=1)
        ss = jnp.sum(h_self * Uh[None, :], axis=1, keepdims=True)
        ns = jnp.sum(nei3 * Vh[None, None, :], axis=2)
        e = ss + ns
        e = jnp.where(e >= 0, e, 0.2 * e)
        m = jnp.max(e, axis=1, keepdims=True)
        p = jnp.exp(e - m)
        alpha = p / jnp.sum(p, axis=1, keepdims=True)
        agg = jnp.sum(alpha[:, :, None] * nei3, axis=1)
        t = jnp.dot(agg, Wnei[h], preferred_element_type=jnp.float32)
        outs.append(jnp.where(t > 0, t, jnp.exp(t) - 1.0))
    return jnp.concatenate(outs, axis=1)


def _tc_user(h_user, rows_un, Wself_u, Wnei_u, a_u, W_u, b_u):
    NBLK = 128
    grid = (B // NBLK,)

    def body(hu_ref, nei_ref, ws_ref, wn_ref, a_ref, wu_ref, bu_ref, out_ref):
        nei3 = nei_ref[...].reshape(NBLK, DEG, D)
        cat = _attend(hu_ref[...], nei3, ws_ref[...], wn_ref[...], a_ref[...])
        out_ref[...] = (
            jnp.dot(cat, wu_ref[...], preferred_element_type=jnp.float32)
            + bu_ref[...]
        )

    return pl.pallas_call(
        body,
        grid=grid,
        in_specs=[
            pl.BlockSpec((NBLK, D), lambda i: (i, 0)),
            pl.BlockSpec((NBLK * DEG, D), lambda i: (i, 0)),
            pl.BlockSpec((H, D, OUT), lambda i: (0, 0, 0)),
            pl.BlockSpec((H, D, OUT), lambda i: (0, 0, 0)),
            pl.BlockSpec((H, 2 * OUT), lambda i: (0, 0)),
            pl.BlockSpec((H * OUT, OUT), lambda i: (0, 0)),
            pl.BlockSpec((1, OUT), lambda i: (0, 0)),
        ],
        out_specs=pl.BlockSpec((NBLK, OUT), lambda i: (i, 0)),
        out_shape=jax.ShapeDtypeStruct((B, OUT), jnp.float32),
    )(h_user, rows_un, Wself_u, Wnei_u, a_u, W_u, b_u)


def _tc_recipe(remb_pad, rows_ru, rows_rr, rows_ri,
               Wself_R, Wnei_R, a_R, W_r, b_r, W_rel, q_rel, H_U):
    NBLK = 128
    grid = (NR_PAD // NBLK,)

    def body(rs_ref, ru_ref, rr_ref, ri_ref, ws_ref, wn_ref, a_ref,
             wr_ref, br_ref, wrel_ref, qrel_ref, hu_ref, out_ref):
        h_self = rs_ref[...]
        rels = []
        for r, nref in enumerate((ru_ref, rr_ref, ri_ref)):
            nei3 = nref[...].reshape(NBLK, DEG, D)
            cat = _attend(h_self, nei3, ws_ref[r], wn_ref[r], a_ref[r])
            rels.append(
                jnp.dot(cat, wr_ref[...], preferred_element_type=jnp.float32)
                + br_ref[...]
            )
        ws = []
        for r in range(3):
            t = jnp.tanh(jnp.dot(rels[r], wrel_ref[...],
                                 preferred_element_type=jnp.float32))
            ws.append(jnp.sum(t * qrel_ref[...], axis=1, keepdims=True))
        m = jnp.maximum(jnp.maximum(ws[0], ws[1]), ws[2])
        e0 = jnp.exp(ws[0] - m)
        e1 = jnp.exp(ws[1] - m)
        e2 = jnp.exp(ws[2] - m)
        denom = e0 + e1 + e2
        H_R = (e0 * rels[0] + e1 * rels[1] + e2 * rels[2]) / denom
        out_ref[...] = lax.dot_general(
            hu_ref[...], H_R, (((1,), (1,)), ((), ())),
            preferred_element_type=jnp.float32)

    return pl.pallas_call(
        body,
        grid=grid,
        in_specs=[
            pl.BlockSpec((NBLK, D), lambda j: (j, 0)),
            pl.BlockSpec((NBLK * DEG, D), lambda j: (j, 0)),
            pl.BlockSpec((NBLK * DEG, D), lambda j: (j, 0)),
            pl.BlockSpec((NBLK * DEG, D), lambda j: (j, 0)),
            pl.BlockSpec((3, H, D, OUT), lambda j: (0, 0, 0, 0)),
            pl.BlockSpec((3, H, D, OUT), lambda j: (0, 0, 0, 0)),
            pl.BlockSpec((3, H, 2 * OUT), lambda j: (0, 0, 0)),
            pl.BlockSpec((H * OUT, OUT), lambda j: (0, 0)),
            pl.BlockSpec((1, OUT), lambda j: (0, 0)),
            pl.BlockSpec((OUT, OUT), lambda j: (0, 0)),
            pl.BlockSpec((1, OUT), lambda j: (0, 0)),
            pl.BlockSpec((B, OUT), lambda j: (0, 0)),
        ],
        out_specs=pl.BlockSpec((B, NBLK), lambda j: (0, j)),
        out_shape=jax.ShapeDtypeStruct((B, NR_PAD), jnp.float32),
    )(remb_pad, rows_ru, rows_rr, rows_ri,
      Wself_R, Wnei_R, a_R, W_r, b_r, W_rel, q_rel, H_U)


def kernel(user, item_seq, user_recipe_idx, recipe_user_idx, recipe_recipe_idx,
           recipe_ing_idx, user_embedding, recipe_embedding, ing_embedding,
           Wself_u, Wnei_u, a_u, Wself_ru, Wnei_ru, a_ru, Wself_rr, Wnei_rr,
           a_rr, Wself_ri, Wnei_ri, a_ri, W_u, b_u, W_r, b_r, W_rel, q_rel):
    del item_seq
    pad_e = E_REL - N_RECIPE * DEG
    idx_ru = jnp.pad(recipe_user_idx.reshape(-1).astype(jnp.int32), (0, pad_e))
    idx_rr = jnp.pad(recipe_recipe_idx.reshape(-1).astype(jnp.int32), (0, pad_e))
    idx_ri = jnp.pad(recipe_ing_idx.reshape(-1).astype(jnp.int32), (0, pad_e))

    uri_pad = jnp.pad(user_recipe_idx.astype(jnp.int32), ((0, 0), (0, 128 - DEG)))
    unei_idx, h_user = _sc_nested(user.astype(jnp.int32), uri_pad, user_embedding)
    rows_ru, rows_rr, rows_ri, rows_un = _sc_gather_rows(
        idx_ru, idx_rr, idx_ri, unei_idx[:, :DEG].reshape(-1),
        user_embedding, recipe_embedding, ing_embedding)

    H_U = _tc_user(h_user, rows_un, Wself_u, Wnei_u, a_u,
                   W_u, b_u.reshape(1, OUT))

    remb_pad = jnp.pad(recipe_embedding, ((0, NR_PAD - N_RECIPE), (0, 0)))
    Wself_R = jnp.stack([Wself_ru, Wself_rr, Wself_ri])
    Wnei_R = jnp.stack([Wnei_ru, Wnei_rr, Wnei_ri])
    a_R = jnp.stack([a_ru, a_rr, a_ri])

    pred_pad = _tc_recipe(remb_pad, rows_ru, rows_rr, rows_ri,
                          Wself_R, Wnei_R, a_R, W_r, b_r.reshape(1, OUT),
                          W_rel, q_rel.reshape(1, OUT), H_U)
    return pred_pad[:, :N_RECIPE]


# speedup vs baseline: 1.5725x; 1.0791x over previous
import functools

import jax
import jax.numpy as jnp
from jax import lax
from jax.experimental import pallas as pl
from jax.experimental.pallas import tpu as pltpu
from jax.experimental.pallas import tpu_sc as plsc

N_USER = 10000
N_RECIPE = 5900
N_ING = 6000
D = 128
H = 4
OUT = 128
B = 1024
DEG = 16

NR_PAD = 6144
E_REL = NR_PAD * DEG
E_U = B * DEG

_NC, _NS = 2, 16
_NW = _NC * _NS
_CH = 128


def _widx():
    return lax.axis_index("s") * _NC + lax.axis_index("c")


def _sc_nested(user, user_recipe_idx_pad, user_embedding):
    per_w = B // _NW

    mesh = plsc.VectorSubcoreMesh(core_axis_name="c", subcore_axis_name="s")

    @functools.partial(
        pl.kernel,
        out_type=(
            jax.ShapeDtypeStruct((B, 128), jnp.int32),
            jax.ShapeDtypeStruct((B, D), jnp.float32),
        ),
        mesh=mesh,
        scratch_types=[
            pltpu.VMEM((per_w,), jnp.int32),
            pltpu.VMEM((per_w, 128), jnp.int32),
            pltpu.VMEM((per_w, D), jnp.float32),
            pltpu.SemaphoreType.DMA,
            pltpu.SemaphoreType.DMA,
        ],
    )
    def k(user_h, uri_h, uemb_h, idx_out_h, hu_out_h, uid_v, urow_v, emb_v, s1, s2):
        wid = _widx()
        base = wid * per_w
        pltpu.sync_copy(user_h.at[pl.ds(base, per_w)], uid_v)
        c1 = pltpu.async_copy(uri_h.at[uid_v], urow_v, s1)
        c2 = pltpu.async_copy(uemb_h.at[uid_v], emb_v, s2)
        c1.wait()
        pltpu.sync_copy(urow_v, idx_out_h.at[pl.ds(base, per_w)])
        c2.wait()
        pltpu.sync_copy(emb_v, hu_out_h.at[pl.ds(base, per_w)])

    return k(user, user_recipe_idx_pad, user_embedding)


_SB = 3
_IR_REL = E_REL // _NW // _CH
_IR_UN = E_U // _NW // _CH


def _sc_gather_rows(idx_ru, idx_rr, idx_ri, idx_un,
                    user_embedding, recipe_embedding, ing_embedding):
    mesh = plsc.VectorSubcoreMesh(core_axis_name="c", subcore_axis_name="s")

    @functools.partial(
        pl.kernel,
        out_type=(
            jax.ShapeDtypeStruct((E_REL, D), jnp.float32),
            jax.ShapeDtypeStruct((E_REL, D), jnp.float32),
            jax.ShapeDtypeStruct((E_REL, D), jnp.float32),
            jax.ShapeDtypeStruct((E_U, D), jnp.float32),
        ),
        mesh=mesh,
        scratch_types=[
            pltpu.VMEM((3 * _IR_REL + _IR_UN, _CH), jnp.int32),
            pltpu.VMEM((_SB * _CH, D), jnp.float32),
            pltpu.VMEM((_SB * _CH, D), jnp.float32),
            pltpu.SemaphoreType.DMA,
            pltpu.SemaphoreType.DMA,
            pltpu.SemaphoreType.DMA,
            pltpu.SemaphoreType.DMA,
        ],
    )
    def k(iru_h, irr_h, iri_h, iun_h, uemb_h, remb_h, iemb_h,
          oru_h, orr_h, ori_h, oun_h, idx_v, r0, r1, g0, g1, w0, w1):
        wid = _widx()
        rows = (r0, r1)
        gsem = (g0, g1)
        wsem = (w0, w1)

        tasks = [
            (iru_h, uemb_h, oru_h, _IR_REL, 0),
            (irr_h, remb_h, orr_h, _IR_REL, _IR_REL),
            (iri_h, iemb_h, ori_h, _IR_REL, 2 * _IR_REL),
            (iun_h, remb_h, oun_h, _IR_UN, 3 * _IR_REL),
        ]
        for idx_h, _, _, n_ir, off in tasks:
            pltpu.sync_copy(idx_h.at[pl.ds(wid * n_ir, n_ir)],
                            idx_v.at[pl.ds(off, n_ir)])

        pending = [None, None]
        step = 0
        for idx_h, table_h, out_h, n_ir, off in tasks:
            nsb = -(-n_ir // _SB)
            for c in range(nsb):
                sb = min(_SB, n_ir - c * _SB)
                b = step % 2
                if pending[b] is not None:
                    pending[b].wait()
                    pending[b] = None
                copies = []
                for j in range(sb):
                    ir = off + c * _SB + j
                    copies.append(pltpu.async_copy(
                        table_h.at[idx_v.at[ir]],
                        rows[b].at[pl.ds(j * _CH, _CH)], gsem[b]))
                for cp in copies:
                    cp.wait()
                base = wid * n_ir * _CH + c * _SB * _CH
                pending[b] = pltpu.async_copy(
                    rows[b].at[pl.ds(0, sb * _CH)],
                    out_h.at[pl.ds(base, sb * _CH)], wsem[b])
                step += 1
        for b in range(2):
            if pending[b] is not None:
                pending[b].wait()

    return k(idx_ru, idx_rr, idx_ri, idx_un,
             user_embedding, recipe_embedding, ing_embedding)


def _attend(h_self, nei3, Wself, Wnei, a):
    outs = []
    for h in range(H):
        a1 = a[h:h + 1, :OUT]
        a2 = a[h:h + 1, OUT:]
        Uh = jnp.sum(Wself[h] * a1, axis=1)
        Vh = jnp.sum(Wnei[h] * a2, axis=1)
        ss = jnp.sum(h_self * Uh[None, :], axis=1, keepdims=True)
        ns = jnp.sum(nei3 * Vh[None, None, :], axis=2)
        e = ss + ns
        e = jnp.where(e >= 0, e, 0.2 * e)
        m = jnp.max(e, axis=1, keepdims=True)
        p = jnp.exp(e - m)
        alpha = p / jnp.sum(p, axis=1, keepdims=True)
        agg = jnp.sum(alpha[:, :, None] * nei3, axis=1)
        t = jnp.dot(agg, Wnei[h], preferred_element_type=jnp.float32)
        outs.append(jnp.where(t > 0, t, jnp.exp(t) - 1.0))
    return jnp.concatenate(outs, axis=1)


def _tc_user(h_user, rows_un, Wself_u, Wnei_u, a_u, W_u, b_u):
    NBLK = 128
    grid = (B // NBLK,)

    def body(hu_ref, nei_ref, ws_ref, wn_ref, a_ref, wu_ref, bu_ref, out_ref):
        nei3 = nei_ref[...].reshape(NBLK, DEG, D)
        cat = _attend(hu_ref[...], nei3, ws_ref[...], wn_ref[...], a_ref[...])
        out_ref[...] = (
            jnp.dot(cat, wu_ref[...], preferred_element_type=jnp.float32)
            + bu_ref[...]
        )

    return pl.pallas_call(
        body,
        grid=grid,
        in_specs=[
            pl.BlockSpec((NBLK, D), lambda i: (i, 0)),
            pl.BlockSpec((NBLK * DEG, D), lambda i: (i, 0)),
            pl.BlockSpec((H, D, OUT), lambda i: (0, 0, 0)),
            pl.BlockSpec((H, D, OUT), lambda i: (0, 0, 0)),
            pl.BlockSpec((H, 2 * OUT), lambda i: (0, 0)),
            pl.BlockSpec((H * OUT, OUT), lambda i: (0, 0)),
            pl.BlockSpec((1, OUT), lambda i: (0, 0)),
        ],
        out_specs=pl.BlockSpec((NBLK, OUT), lambda i: (i, 0)),
        out_shape=jax.ShapeDtypeStruct((B, OUT), jnp.float32),
    )(h_user, rows_un, Wself_u, Wnei_u, a_u, W_u, b_u)


def _tc_recipe(remb_pad, rows_ru, rows_rr, rows_ri,
               Wself_R, Wnei_R, a_R, W_r, b_r, W_rel, q_rel, H_U):
    NBLK = 128
    grid = (NR_PAD // NBLK,)

    def body(rs_ref, ru_ref, rr_ref, ri_ref, ws_ref, wn_ref, a_ref,
             wr_ref, br_ref, wrel_ref, qrel_ref, hu_ref, out_ref):
        h_self = rs_ref[...]
        rels = []
        for r, nref in enumerate((ru_ref, rr_ref, ri_ref)):
            nei3 = nref[...].reshape(NBLK, DEG, D)
            cat = _attend(h_self, nei3, ws_ref[r], wn_ref[r], a_ref[r])
            rels.append(
                jnp.dot(cat, wr_ref[...], preferred_element_type=jnp.float32)
                + br_ref[...]
            )
        ws = []
        for r in range(3):
            t = jnp.tanh(jnp.dot(rels[r], wrel_ref[...],
                                 preferred_element_type=jnp.float32))
            ws.append(jnp.sum(t * qrel_ref[...], axis=1, keepdims=True))
        m = jnp.maximum(jnp.maximum(ws[0], ws[1]), ws[2])
        e0 = jnp.exp(ws[0] - m)
        e1 = jnp.exp(ws[1] - m)
        e2 = jnp.exp(ws[2] - m)
        denom = e0 + e1 + e2
        H_R = (e0 * rels[0] + e1 * rels[1] + e2 * rels[2]) / denom
        out_ref[...] = lax.dot_general(
            hu_ref[...], H_R, (((1,), (1,)), ((), ())),
            preferred_element_type=jnp.float32)

    return pl.pallas_call(
        body,
        grid=grid,
        in_specs=[
            pl.BlockSpec((NBLK, D), lambda j: (j, 0)),
            pl.BlockSpec((NBLK * DEG, D), lambda j: (j, 0)),
            pl.BlockSpec((NBLK * DEG, D), lambda j: (j, 0)),
            pl.BlockSpec((NBLK * DEG, D), lambda j: (j, 0)),
            pl.BlockSpec((3, H, D, OUT), lambda j: (0, 0, 0, 0)),
            pl.BlockSpec((3, H, D, OUT), lambda j: (0, 0, 0, 0)),
            pl.BlockSpec((3, H, 2 * OUT), lambda j: (0, 0, 0)),
            pl.BlockSpec((H * OUT, OUT), lambda j: (0, 0)),
            pl.BlockSpec((1, OUT), lambda j: (0, 0)),
            pl.BlockSpec((OUT, OUT), lambda j: (0, 0)),
            pl.BlockSpec((1, OUT), lambda j: (0, 0)),
            pl.BlockSpec((B, OUT), lambda j: (0, 0)),
        ],
        out_specs=pl.BlockSpec((B, NBLK), lambda j: (0, j)),
        out_shape=jax.ShapeDtypeStruct((B, NR_PAD), jnp.float32),
    )(remb_pad, rows_ru, rows_rr, rows_ri,
      Wself_R, Wnei_R, a_R, W_r, b_r, W_rel, q_rel, H_U)


def kernel(user, item_seq, user_recipe_idx, recipe_user_idx, recipe_recipe_idx,
           recipe_ing_idx, user_embedding, recipe_embedding, ing_embedding,
           Wself_u, Wnei_u, a_u, Wself_ru, Wnei_ru, a_ru, Wself_rr, Wnei_rr,
           a_rr, Wself_ri, Wnei_ri, a_ri, W_u, b_u, W_r, b_r, W_rel, q_rel):
    del item_seq
    pad_e = E_REL - N_RECIPE * DEG
    idx_ru = jnp.pad(recipe_user_idx.reshape(-1).astype(jnp.int32), (0, pad_e))
    idx_rr = jnp.pad(recipe_recipe_idx.reshape(-1).astype(jnp.int32), (0, pad_e))
    idx_ri = jnp.pad(recipe_ing_idx.reshape(-1).astype(jnp.int32), (0, pad_e))

    uri_pad = jnp.pad(user_recipe_idx.astype(jnp.int32), ((0, 0), (0, 128 - DEG)))
    unei_idx, h_user = _sc_nested(user.astype(jnp.int32), uri_pad, user_embedding)
    rows_ru, rows_rr, rows_ri, rows_un = _sc_gather_rows(
        idx_ru.reshape(E_REL // _CH, _CH),
        idx_rr.reshape(E_REL // _CH, _CH),
        idx_ri.reshape(E_REL // _CH, _CH),
        unei_idx[:, :DEG].reshape(E_U // _CH, _CH),
        user_embedding, recipe_embedding, ing_embedding)

    H_U = _tc_user(h_user, rows_un, Wself_u, Wnei_u, a_u,
                   W_u, b_u.reshape(1, OUT))

    remb_pad = jnp.pad(recipe_embedding, ((0, NR_PAD - N_RECIPE), (0, 0)))
    Wself_R = jnp.stack([Wself_ru, Wself_rr, Wself_ri])
    Wnei_R = jnp.stack([Wnei_ru, Wnei_rr, Wnei_ri])
    a_R = jnp.stack([a_ru, a_rr, a_ri])

    pred_pad = _tc_recipe(remb_pad, rows_ru, rows_rr, rows_ri,
                          Wself_R, Wnei_R, a_R, W_r, b_r.reshape(1, OUT),
                          W_rel, q_rel.reshape(1, OUT), H_U)
    return pred_pad[:, :N_RECIPE]


# speedup vs baseline: 1.6251x; 1.0334x over previous
import functools

import jax
import jax.numpy as jnp
from jax import lax
from jax.experimental import pallas as pl
from jax.experimental.pallas import tpu as pltpu
from jax.experimental.pallas import tpu_sc as plsc

N_USER = 10000
N_RECIPE = 5900
N_ING = 6000
D = 128
H = 4
OUT = 128
B = 1024
DEG = 16

NR_PAD = 6144
E_REL = NR_PAD * DEG
E_U = B * DEG

_NC, _NS = 2, 16
_NW = _NC * _NS
_CH = 128


def _widx():
    return lax.axis_index("s") * _NC + lax.axis_index("c")


def _sc_nested(user, user_recipe_idx_pad, user_embedding):
    per_w = B // _NW

    mesh = plsc.VectorSubcoreMesh(core_axis_name="c", subcore_axis_name="s")

    @functools.partial(
        pl.kernel,
        out_type=(
            jax.ShapeDtypeStruct((B, 128), jnp.int32),
            jax.ShapeDtypeStruct((B, D), jnp.float32),
        ),
        mesh=mesh,
        scratch_types=[
            pltpu.VMEM((per_w,), jnp.int32),
            pltpu.VMEM((per_w, 128), jnp.int32),
            pltpu.VMEM((per_w, D), jnp.float32),
            pltpu.SemaphoreType.DMA,
            pltpu.SemaphoreType.DMA,
        ],
    )
    def k(user_h, uri_h, uemb_h, idx_out_h, hu_out_h, uid_v, urow_v, emb_v, s1, s2):
        wid = _widx()
        base = wid * per_w
        pltpu.sync_copy(user_h.at[pl.ds(base, per_w)], uid_v)
        c1 = pltpu.async_copy(uri_h.at[uid_v], urow_v, s1)
        c2 = pltpu.async_copy(uemb_h.at[uid_v], emb_v, s2)
        c1.wait()
        pltpu.sync_copy(urow_v, idx_out_h.at[pl.ds(base, per_w)])
        c2.wait()
        pltpu.sync_copy(emb_v, hu_out_h.at[pl.ds(base, per_w)])

    return k(user, user_recipe_idx_pad, user_embedding)


_SB = 256
_NSB_REL = E_REL // _NW // _SB
_NSB_UN = E_U // _NW // _SB
_NBUF = 3


def _sc_gather_rows(idx_ru, idx_rr, idx_ri, idx_un,
                    user_embedding, recipe_embedding, ing_embedding):
    mesh = plsc.VectorSubcoreMesh(core_axis_name="c", subcore_axis_name="s")

    nsb_tot = 3 * _NSB_REL + _NSB_UN

    @functools.partial(
        pl.kernel,
        out_type=(
            jax.ShapeDtypeStruct((E_REL, D), jnp.float32),
            jax.ShapeDtypeStruct((E_REL, D), jnp.float32),
            jax.ShapeDtypeStruct((E_REL, D), jnp.float32),
            jax.ShapeDtypeStruct((E_U, D), jnp.float32),
        ),
        mesh=mesh,
        scratch_types=[
            pltpu.VMEM((nsb_tot * _SB,), jnp.int32),
            pltpu.VMEM((_SB, D), jnp.float32),
            pltpu.VMEM((_SB, D), jnp.float32),
            pltpu.VMEM((_SB, D), jnp.float32),
            pltpu.SemaphoreType.DMA,
            pltpu.SemaphoreType.DMA,
            pltpu.SemaphoreType.DMA,
            pltpu.SemaphoreType.DMA,
            pltpu.SemaphoreType.DMA,
            pltpu.SemaphoreType.DMA,
        ],
    )
    def k(iru_h, irr_h, iri_h, iun_h, uemb_h, remb_h, iemb_h,
          oru_h, orr_h, ori_h, oun_h, idx_v, r0, r1, r2,
          g0, g1, g2, w0, w1, w2):
        wid = _widx()
        rows = (r0, r1, r2)
        gsem = (g0, g1, g2)
        wsem = (w0, w1, w2)

        tasks = [
            (iru_h, uemb_h, oru_h, _NSB_REL, 0),
            (irr_h, remb_h, orr_h, _NSB_REL, _NSB_REL),
            (iri_h, iemb_h, ori_h, _NSB_REL, 2 * _NSB_REL),
            (iun_h, remb_h, oun_h, _NSB_UN, 3 * _NSB_REL),
        ]
        for idx_h, _, _, nsb, off in tasks:
            pltpu.sync_copy(idx_h.at[pl.ds(wid * nsb * _SB, nsb * _SB)],
                            idx_v.at[pl.ds(off * _SB, nsb * _SB)])

        work = []
        for idx_h, table_h, out_h, nsb, off in tasks:
            for c in range(nsb):
                work.append((table_h, out_h, nsb, off, c))

        pend_g = [None] * _NBUF
        pend_w = [None] * _NBUF

        def issue(s):
            if s >= len(work):
                return
            b = s % _NBUF
            table_h, out_h, nsb, off, c = work[s]
            if pend_w[b] is not None:
                pend_w[b].wait()
                pend_w[b] = None
            pend_g[b] = pltpu.async_copy(
                table_h.at[idx_v.at[pl.ds((off + c) * _SB, _SB)]],
                rows[b], gsem[b])

        for s in range(_NBUF - 1):
            issue(s)
        for s in range(len(work)):
            b = s % _NBUF
            issue(s + _NBUF - 1)
            table_h, out_h, nsb, off, c = work[s]
            pend_g[b].wait()
            pend_g[b] = None
            base = (wid * nsb + c) * _SB
            pend_w[b] = pltpu.async_copy(
                rows[b], out_h.at[pl.ds(base, _SB)], wsem[b])
        for b in range(_NBUF):
            if pend_w[b] is not None:
                pend_w[b].wait()

    return k(idx_ru, idx_rr, idx_ri, idx_un,
             user_embedding, recipe_embedding, ing_embedding)


def _attend(h_self, nei3, Wself, Wnei, a):
    outs = []
    for h in range(H):
        a1 = a[h:h + 1, :OUT]
        a2 = a[h:h + 1, OUT:]
        Uh = jnp.sum(Wself[h] * a1, axis=1)
        Vh = jnp.sum(Wnei[h] * a2, axis=1)
        ss = jnp.sum(h_self * Uh[None, :], axis=1, keepdims=True)
        ns = jnp.sum(nei3 * Vh[None, None, :], axis=2)
        e = ss + ns
        e = jnp.where(e >= 0, e, 0.2 * e)
        m = jnp.max(e, axis=1, keepdims=True)
        p = jnp.exp(e - m)
        alpha = p / jnp.sum(p, axis=1, keepdims=True)
        agg = jnp.sum(alpha[:, :, None] * nei3, axis=1)
        t = jnp.dot(agg, Wnei[h], preferred_element_type=jnp.float32)
        outs.append(jnp.where(t > 0, t, jnp.exp(t) - 1.0))
    return jnp.concatenate(outs, axis=1)


def _tc_user(h_user, rows_un, Wself_u, Wnei_u, a_u, W_u, b_u):
    NBLK = 128
    grid = (B // NBLK,)

    def body(hu_ref, nei_ref, ws_ref, wn_ref, a_ref, wu_ref, bu_ref, out_ref):
        nei3 = nei_ref[...].reshape(NBLK, DEG, D)
        cat = _attend(hu_ref[...], nei3, ws_ref[...], wn_ref[...], a_ref[...])
        out_ref[...] = (
            jnp.dot(cat, wu_ref[...], preferred_element_type=jnp.float32)
            + bu_ref[...]
        )

    return pl.pallas_call(
        body,
        grid=grid,
        in_specs=[
            pl.BlockSpec((NBLK, D), lambda i: (i, 0)),
            pl.BlockSpec((NBLK * DEG, D), lambda i: (i, 0)),
            pl.BlockSpec((H, D, OUT), lambda i: (0, 0, 0)),
            pl.BlockSpec((H, D, OUT), lambda i: (0, 0, 0)),
            pl.BlockSpec((H, 2 * OUT), lambda i: (0, 0)),
            pl.BlockSpec((H * OUT, OUT), lambda i: (0, 0)),
            pl.BlockSpec((1, OUT), lambda i: (0, 0)),
        ],
        out_specs=pl.BlockSpec((NBLK, OUT), lambda i: (i, 0)),
        out_shape=jax.ShapeDtypeStruct((B, OUT), jnp.float32),
    )(h_user, rows_un, Wself_u, Wnei_u, a_u, W_u, b_u)


def _tc_recipe(remb_pad, rows_ru, rows_rr, rows_ri,
               Wself_R, Wnei_R, a_R, W_r, b_r, W_rel, q_rel, H_U):
    NBLK = 128
    grid = (NR_PAD // NBLK,)

    def body(rs_ref, ru_ref, rr_ref, ri_ref, ws_ref, wn_ref, a_ref,
             wr_ref, br_ref, wrel_ref, qrel_ref, hu_ref, out_ref):
        h_self = rs_ref[...]
        rels = []
        for r, nref in enumerate((ru_ref, rr_ref, ri_ref)):
            nei3 = nref[...].reshape(NBLK, DEG, D)
            cat = _attend(h_self, nei3, ws_ref[r], wn_ref[r], a_ref[r])
            rels.append(
                jnp.dot(cat, wr_ref[...], preferred_element_type=jnp.float32)
                + br_ref[...]
            )
        ws = []
        for r in range(3):
            t = jnp.tanh(jnp.dot(rels[r], wrel_ref[...],
                                 preferred_element_type=jnp.float32))
            ws.append(jnp.sum(t * qrel_ref[...], axis=1, keepdims=True))
        m = jnp.maximum(jnp.maximum(ws[0], ws[1]), ws[2])
        e0 = jnp.exp(ws[0] - m)
        e1 = jnp.exp(ws[1] - m)
        e2 = jnp.exp(ws[2] - m)
        denom = e0 + e1 + e2
        H_R = (e0 * rels[0] + e1 * rels[1] + e2 * rels[2]) / denom
        out_ref[...] = lax.dot_general(
            hu_ref[...], H_R, (((1,), (1,)), ((), ())),
            preferred_element_type=jnp.float32)

    return pl.pallas_call(
        body,
        grid=grid,
        in_specs=[
            pl.BlockSpec((NBLK, D), lambda j: (j, 0)),
            pl.BlockSpec((NBLK * DEG, D), lambda j: (j, 0)),
            pl.BlockSpec((NBLK * DEG, D), lambda j: (j, 0)),
            pl.BlockSpec((NBLK * DEG, D), lambda j: (j, 0)),
            pl.BlockSpec((3, H, D, OUT), lambda j: (0, 0, 0, 0)),
            pl.BlockSpec((3, H, D, OUT), lambda j: (0, 0, 0, 0)),
            pl.BlockSpec((3, H, 2 * OUT), lambda j: (0, 0, 0)),
            pl.BlockSpec((H * OUT, OUT), lambda j: (0, 0)),
            pl.BlockSpec((1, OUT), lambda j: (0, 0)),
            pl.BlockSpec((OUT, OUT), lambda j: (0, 0)),
            pl.BlockSpec((1, OUT), lambda j: (0, 0)),
            pl.BlockSpec((B, OUT), lambda j: (0, 0)),
        ],
        out_specs=pl.BlockSpec((B, NBLK), lambda j: (0, j)),
        out_shape=jax.ShapeDtypeStruct((B, NR_PAD), jnp.float32),
    )(remb_pad, rows_ru, rows_rr, rows_ri,
      Wself_R, Wnei_R, a_R, W_r, b_r, W_rel, q_rel, H_U)


def kernel(user, item_seq, user_recipe_idx, recipe_user_idx, recipe_recipe_idx,
           recipe_ing_idx, user_embedding, recipe_embedding, ing_embedding,
           Wself_u, Wnei_u, a_u, Wself_ru, Wnei_ru, a_ru, Wself_rr, Wnei_rr,
           a_rr, Wself_ri, Wnei_ri, a_ri, W_u, b_u, W_r, b_r, W_rel, q_rel):
    del item_seq
    pad_e = E_REL - N_RECIPE * DEG
    idx_ru = jnp.pad(recipe_user_idx.reshape(-1).astype(jnp.int32), (0, pad_e))
    idx_rr = jnp.pad(recipe_recipe_idx.reshape(-1).astype(jnp.int32), (0, pad_e))
    idx_ri = jnp.pad(recipe_ing_idx.reshape(-1).astype(jnp.int32), (0, pad_e))

    uri_pad = jnp.pad(user_recipe_idx.astype(jnp.int32), ((0, 0), (0, 128 - DEG)))
    unei_idx, h_user = _sc_nested(user.astype(jnp.int32), uri_pad, user_embedding)
    rows_ru, rows_rr, rows_ri, rows_un = _sc_gather_rows(
        idx_ru, idx_rr, idx_ri, unei_idx[:, :DEG].reshape(E_U),
        user_embedding, recipe_embedding, ing_embedding)

    H_U = _tc_user(h_user, rows_un, Wself_u, Wnei_u, a_u,
                   W_u, b_u.reshape(1, OUT))

    remb_pad = jnp.pad(recipe_embedding, ((0, NR_PAD - N_RECIPE), (0, 0)))
    Wself_R = jnp.stack([Wself_ru, Wself_rr, Wself_ri])
    Wnei_R = jnp.stack([Wnei_ru, Wnei_rr, Wnei_ri])
    a_R = jnp.stack([a_ru, a_rr, a_ri])

    pred_pad = _tc_recipe(remb_pad, rows_ru, rows_rr, rows_ri,
                          Wself_R, Wnei_R, a_R, W_r, b_r.reshape(1, OUT),
                          W_rel, q_rel.reshape(1, OUT), H_U)
    return pred_pad[:, :N_RECIPE]
